# baseline (speedup 1.0000x reference)
import functools

import jax
import jax.numpy as jnp
from jax.experimental import pallas as pl
from jax.experimental.pallas import tpu as pltpu

_BM = 256
_BK = 1024


def _vq_loss_kernel(z_ref, ct_ref, out_ref, csq_ref, *, n_tokens, k_codes):
    @pl.when(pl.program_id(0) == 0)
    def _csq():
        ct = ct_ref[:]
        csq_ref[:] = jnp.sum(ct * ct, axis=0, keepdims=True)

    zb = z_ref[:]
    zb2 = -2.0 * zb

    def body(i, minacc):
        cb = ct_ref[:, pl.ds(i * _BK, _BK)]
        dots = jnp.dot(zb2, cb, preferred_element_type=jnp.float32)
        return jnp.minimum(minacc, dots + csq_ref[0, pl.ds(i * _BK, _BK)][None, :])

    minacc = jax.lax.fori_loop(
        0, k_codes // _BK, body,
        jnp.full((zb.shape[0], _BK), jnp.inf, dtype=jnp.float32))
    minv = jnp.min(minacc, axis=1, keepdims=True)
    zsq = jnp.sum(zb * zb, axis=1, keepdims=True)
    s = jnp.sum(minv + zsq)

    @pl.when(pl.program_id(0) == 0)
    def _init():
        out_ref[0, 0] = 0.0

    out_ref[0, 0] += s / n_tokens


def kernel(z, codebook):
    n, d = z.shape
    k = codebook.shape[0]
    ct = codebook.T
    out = pl.pallas_call(
        functools.partial(_vq_loss_kernel, n_tokens=n, k_codes=k),
        grid=(n // _BM,),
        in_specs=[
            pl.BlockSpec((_BM, d), lambda m: (m, 0)),
            pl.BlockSpec((d, k), lambda m: (0, 0)),
        ],
        out_specs=pl.BlockSpec(memory_space=pltpu.SMEM),
        out_shape=jax.ShapeDtypeStruct((1, 1), jnp.float32),
        scratch_shapes=[pltpu.VMEM((1, k), jnp.float32)],
    )(z, ct)
    return out[0, 0]


# speedup vs baseline: 1.3960x; 1.3960x over previous
import functools

import jax
import jax.numpy as jnp
from jax.experimental import pallas as pl
from jax.experimental.pallas import tpu as pltpu

_BM = 256
_BK = 1024


def _vq_loss_kernel(zt_ref, c_ref, out_ref, c2_ref, *, n_tokens, k_codes):
    d = zt_ref.shape[0]

    @pl.when(pl.program_id(0) == 0)
    def _augment():
        c = c_ref[:]
        c2_ref[:, 0:d] = c
        c2_ref[:, d:d + 1] = jnp.sum(c * c, axis=1, keepdims=True)

    ztb = zt_ref[:]
    z2 = jnp.concatenate(
        [-2.0 * ztb, jnp.ones((1, ztb.shape[1]), jnp.float32)], axis=0)

    def body(i, minv):
        c2 = c2_ref[pl.ds(i * _BK, _BK), :]
        part = jnp.dot(c2, z2, preferred_element_type=jnp.float32)
        return jnp.minimum(minv, jnp.min(part, axis=0, keepdims=True))

    minv = jax.lax.fori_loop(
        0, k_codes // _BK, body,
        jnp.full((1, ztb.shape[1]), jnp.inf, dtype=jnp.float32))
    zsq = jnp.sum(ztb * ztb, axis=0, keepdims=True)
    s = jnp.sum(minv + zsq)

    @pl.when(pl.program_id(0) == 0)
    def _init():
        out_ref[0, 0] = 0.0

    out_ref[0, 0] += s / n_tokens


def kernel(z, codebook):
    n, d = z.shape
    k = codebook.shape[0]
    zt = z.T
    out = pl.pallas_call(
        functools.partial(_vq_loss_kernel, n_tokens=n, k_codes=k),
        grid=(n // _BM,),
        in_specs=[
            pl.BlockSpec((d, _BM), lambda m: (0, m)),
            pl.BlockSpec((k, d), lambda m: (0, 0)),
        ],
        out_specs=pl.BlockSpec(memory_space=pltpu.SMEM),
        out_shape=jax.ShapeDtypeStruct((1, 1), jnp.float32),
        scratch_shapes=[pltpu.VMEM((k, d + 1), jnp.float32)],
    )(zt, codebook)
    return out[0, 0]
